# baseline (speedup 1.0000x reference)
import functools

import jax
import jax.numpy as jnp
import numpy as np
from jax import lax
from jax.experimental import pallas as pl
from jax.experimental.pallas import tpu as pltpu
from jax.experimental.pallas import tpu_sc as plsc

_S = (16, 32, 64)
_INV_SCALE = (1.0 / 32.0, 1.0 / 16.0, 1.0 / 8.0)
_C = 80
_N = 5000
_NSUB = 16
_VPS = 24
_NVT = _NSUB * _VPS
_ROW = 5 + _C
_BIG = 1 << 30

_ANCH = [
    [(np.float32(3.6), np.float32(2.8)), (np.float32(5.0), np.float32(6.2)),
     (np.float32(11.7), np.float32(10.2))],
    [(np.float32(1.9), np.float32(3.8)), (np.float32(3.9), np.float32(2.9)),
     (np.float32(3.7), np.float32(7.4))],
    [(np.float32(0.8), np.float32(1.0)), (np.float32(1.6), np.float32(2.2)),
     (np.float32(2.1), np.float32(4.4))],
]
_AREA = [[np.float32(a[0] * a[1]) for a in sc] for sc in _ANCH]

_GDN = lax.GatherDimensionNumbers(offset_dims=(), collapsed_slice_dims=(0,),
                                  start_index_map=(0,))


def _shuf(x, perm):
    return lax.gather(x, perm[:, None], _GDN, (1,),
                      mode=lax.GatherScatterMode.PROMISE_IN_BOUNDS)


def _vmin16(x):
    lanes = lax.iota(jnp.int32, 16)
    for sh in (8, 4, 2, 1):
        x = jnp.minimum(x, _shuf(x, (lanes + sh) & 15))
    return x


def _vor16(x):
    lanes = lax.iota(jnp.int32, 16)
    for sh in (8, 4, 2, 1):
        x = x | _shuf(x, (lanes + sh) & 15)
    return x


_LN2 = 0.6931471805599453
_SQRT2 = 1.4142135623730951


def _log1p6(x):
    bits = plsc.bitcast(x, jnp.int32)
    e = ((bits >> 23) & 255) - 127
    m = plsc.bitcast((bits & 0x007FFFFF) | 0x3F800000, jnp.float32)
    big = m > _SQRT2
    m = jnp.where(big, m * 0.5, m)
    ef = (e + jnp.where(big, 1, 0)).astype(jnp.float32)
    r = (m - 1.0) / (m + 1.0)
    r2 = r * r
    p = 1.0 + r2 * (1.0 / 3.0 + r2 * (0.2 + r2 * (1.0 / 7.0 + r2 * (1.0 / 9.0))))
    return 2.0 * r * p + ef * _LN2


_info = plsc.get_sparse_core_info()
_MESH = plsc.VectorSubcoreMesh(core_axis_name="c", subcore_axis_name="s")

_DI = 72
_DF = 36


@functools.partial(
    pl.kernel,
    mesh=_MESH,
    out_type=[jax.ShapeDtypeStruct((_DI,), jnp.int32),
              jax.ShapeDtypeStruct((_DF,), jnp.float32)],
    scratch_types=[
        pltpu.VMEM((_N * 4,), jnp.float32),
        pltpu.VMEM((_N,), jnp.int32),
        pltpu.VMEM((_NVT * 16,), jnp.int32),
        pltpu.VMEM((_NVT,), jnp.int32),
        pltpu.VMEM((_VPS * 16,), jnp.int32),
        pltpu.VMEM((_VPS,), jnp.int32),
        pltpu.VMEM_SHARED((_NVT * 16,), jnp.int32),
        pltpu.VMEM_SHARED((_NVT,), jnp.int32),
        pltpu.VMEM((_DI,), jnp.int32),
        pltpu.VMEM((_DF,), jnp.float32),
        pltpu.SemaphoreType.DMA,
    ],
    compiler_params=pltpu.CompilerParams(needs_layout_passes=False),
)
def _sc_select(bb_hbm, lab_hbm, di_out, df_out,
               vbb, vlab, vmask, vl1, vml, vl1l, shm, shl1, vdi, vdf, sem):
    nc = _info.num_cores
    sub = lax.axis_index("s")
    wid = sub * nc + lax.axis_index("c")
    lanes = lax.iota(jnp.int32, 16)
    zi = jnp.zeros((16,), jnp.int32)

    cps = [pltpu.async_copy(bb_hbm, vbb, sem),
           pltpu.async_copy(lab_hbm, vlab, sem)]
    for c in cps:
        c.wait()

    def mbody(t, _):
        idx = (sub * _VPS + t) * 16 + lanes
        rows = jnp.minimum(idx, _N - 1) * 4
        w = plsc.load_gather(vbb, [rows + 2])
        h = plsc.load_gather(vbb, [rows + 3])
        m = jnp.zeros((16,), jnp.int32)
        for s in range(3):
            bw = w * _INV_SCALE[s]
            bh = h * _INV_SCALE[s]
            area = bw * bh
            for a in range(3):
                aw, ah = _ANCH[s][a]
                inter = jnp.minimum(aw, bw) * jnp.minimum(ah, bh)
                iou = inter / (area + _AREA[s][a] - inter)
                m = m | jnp.where(iou >= 0.2, jnp.int32(1 << (3 * s + a)),
                                  jnp.int32(0))
        m = jnp.where(idx < _N, m, 0)
        vml[pl.ds(t * 16, 16)] = m
        plsc.store_scatter(vl1l, [jnp.minimum(zi + t, _VPS - 1)], _vor16(m),
                           mask=lanes == 0)
        return 0
    lax.fori_loop(0, _VPS, mbody, 0)

    pltpu.sync_copy(vml, shm.at[pl.ds(sub * (_VPS * 16), _VPS * 16)])
    pltpu.sync_copy(vl1l, shl1.at[pl.ds(sub * _VPS, _VPS)])
    plsc.subcore_barrier()
    pltpu.sync_copy(shm, vmask)
    pltpu.sync_copy(shl1, vl1)

    for s in range(3):
        grid = _S[s]
        inv = _INV_SCALE[s]
        sh3 = 3 * s
        allowed = jnp.full((16,), 7, jnp.int32)
        u_prev = jnp.full((16,), -1, jnp.int32)
        for k in range(3):
            def s1(t, acc):
                l1 = vl1[pl.ds(t * 16, 16)]
                jdx = t * 16 + lanes
                cand = (((l1 >> sh3) & allowed) != 0) & \
                       (jdx * 16 + 15 > u_prev)
                return jnp.minimum(acc, jnp.where(cand, jdx, _BIG))
            acc = lax.fori_loop(0, _NVT // 16, s1,
                                jnp.full((16,), _BIG, jnp.int32))
            j1 = _vmin16(acc)
            j1c = jnp.minimum(j1, _NVT - 1)
            gidx = j1c * 16 + lanes
            grp = plsc.load_gather(vmask, [gidx])
            cand = (((grp >> sh3) & allowed) != 0) & (gidx > u_prev)
            u_a = _vmin16(jnp.where(cand, gidx, _BIG))
            def s2(t, acc):
                l1 = vl1[pl.ds(t * 16, 16)]
                jdx = t * 16 + lanes
                cand = (((l1 >> sh3) & allowed) != 0) & (jdx > j1)
                return jnp.minimum(acc, jnp.where(cand, jdx, _BIG))
            acc2 = lax.fori_loop(0, _NVT // 16, s2,
                                 jnp.full((16,), _BIG, jnp.int32))
            j2c = jnp.minimum(_vmin16(acc2), _NVT - 1)
            gidx2 = j2c * 16 + lanes
            grp2 = plsc.load_gather(vmask, [gidx2])
            cand2 = (((grp2 >> sh3) & allowed) != 0) & (gidx2 > u_prev)
            u_b = _vmin16(jnp.where(cand2, gidx2, _BIG))
            u = jnp.where(u_a < _BIG, u_a, u_b)
            found = u < _BIG
            idxv = jnp.minimum(u, _N - 1)

            bw = plsc.load_gather(vbb, [idxv * 4 + 2]) * inv
            bh = plsc.load_gather(vbb, [idxv * 4 + 3]) * inv
            cxs = plsc.load_gather(vbb, [idxv * 4]) * inv
            cys = plsc.load_gather(vbb, [idxv * 4 + 1]) * inv
            labv = plsc.load_gather(vlab, [idxv])
            area = bw * bh
            miou = []
            for a in range(3):
                aw, ah = _ANCH[s][a]
                inter = jnp.minimum(aw, bw) * jnp.minimum(ah, bh)
                iou = inter / (area + _AREA[s][a] - inter)
                al = ((allowed >> a) & 1) != 0
                miou.append(jnp.where(al, iou, 0.0))
            best = jnp.where(miou[1] > miou[0], jnp.int32(1), jnp.int32(0))
            bm = jnp.maximum(miou[0], miou[1])
            best = jnp.where(miou[2] > bm, jnp.int32(2), best)
            best_iou = jnp.maximum(bm, miou[2])
            ig = [((miou[a] > 0.5) & (miou[a] != best_iou)).astype(jnp.int32)
                  for a in range(3)]

            px = cxs.astype(jnp.int32)
            py = cys.astype(jnp.int32)
            bx = cxs - px.astype(jnp.float32)
            by = cys - py.astype(jnp.float32)
            awv = jnp.where(best == 0, _ANCH[s][0][0],
                            jnp.where(best == 1, _ANCH[s][1][0], _ANCH[s][2][0]))
            ahv = jnp.where(best == 0, _ANCH[s][0][1],
                            jnp.where(best == 1, _ANCH[s][1][1], _ANCH[s][2][1]))
            lw = _log1p6(bw / awv + 1e-6)
            lh = _log1p6(bh / ahv + 1e-6)

            vi = jnp.where(
                lanes == 0, found.astype(jnp.int32),
                jnp.where(lanes == 1, px,
                          jnp.where(lanes == 2, py,
                                    jnp.where(lanes == 3, best,
                                              jnp.where(lanes == 4, labv,
                                                        jnp.where(lanes == 5, ig[0],
                                                                  jnp.where(lanes == 6,
                                                                            ig[1], ig[2])))))))
            bi = s * 24 + k * 8
            plsc.store_scatter(vdi, [jnp.minimum(bi + lanes, _DI - 1)], vi,
                               mask=lanes < 8)
            vf = jnp.where(lanes == 0, bx,
                           jnp.where(lanes == 1, by,
                                     jnp.where(lanes == 2, lw, lh)))
            bf = s * 12 + k * 4
            plsc.store_scatter(vdf, [jnp.minimum(bf + lanes, _DF - 1)], vf,
                               mask=lanes < 4)

            bbit = jnp.int32(1) << best
            allowed = jnp.where(found, allowed & ~bbit, allowed)
            u_prev = u

    @pl.when(wid == 0)
    def _():
        pltpu.sync_copy(vdi, di_out)
        pltpu.sync_copy(vdf, df_out)


def _tc_zero(o0, o1, o2):
    for s, o in enumerate((o0, o1, o2)):
        g = _S[s]
        o[...] = jnp.zeros((g, g, 3, _ROW), jnp.float32)


_tc_zero_call = pl.pallas_call(
    _tc_zero,
    out_shape=[jax.ShapeDtypeStruct((g, g, 3, _ROW), jnp.float32) for g in _S],
)


def _tc_apply(di, df, t0, t1, t2, o0, o1, o2, vcell, sem):
    del t0, t1, t2
    for s, o in enumerate((o0, o1, o2)):
        for k in range(3):
            b = s * 24 + k * 8
            fb = s * 12 + k * 4

            @pl.when(di[b] == 1)
            def _(o=o, b=b, fb=fb):
                px = di[b + 1]
                py = di[b + 2]
                bst = di[b + 3]
                lab = di[b + 4]
                cp = pltpu.make_async_copy(o.at[px, py], vcell, sem)
                cp.start()
                cp.wait()
                arow = lax.broadcasted_iota(jnp.int32, (3, _ROW), 0)
                ccol = lax.broadcasted_iota(jnp.int32, (3, _ROW), 1)
                row = jnp.where(
                    ccol == 0, 1.0,
                    jnp.where(ccol == 1, df[fb],
                              jnp.where(ccol == 2, df[fb + 1],
                                        jnp.where(ccol == 3, df[fb + 2],
                                                  jnp.where(ccol == 4, df[fb + 3],
                                                            jnp.where(ccol == 5 + lab,
                                                                      1.0, 0.0))))))
                cell = vcell[...]
                cell = jnp.where(arow == bst, row, cell)
                igm = (((arow == 0) & (di[b + 5] == 1)) |
                       ((arow == 1) & (di[b + 6] == 1)) |
                       ((arow == 2) & (di[b + 7] == 1))) & (ccol == 0)
                vcell[...] = jnp.where(igm, -1.0, cell)
                cp2 = pltpu.make_async_copy(vcell, o.at[px, py], sem)
                cp2.start()
                cp2.wait()


_tc_apply_call = pl.pallas_call(
    _tc_apply,
    out_shape=[jax.ShapeDtypeStruct((g, g, 3, _ROW), jnp.float32) for g in _S],
    in_specs=[pl.BlockSpec(memory_space=pltpu.SMEM),
              pl.BlockSpec(memory_space=pltpu.SMEM)] +
             [pl.BlockSpec(memory_space=pl.ANY)] * 3,
    out_specs=[pl.BlockSpec(memory_space=pl.ANY)] * 3,
    input_output_aliases={2: 0, 3: 1, 4: 2},
    scratch_shapes=[pltpu.VMEM((3, _ROW), jnp.float32),
                    pltpu.SemaphoreType.DMA],
)


def kernel(image, bboxes, labels):
    z0, z1, z2 = _tc_zero_call()
    di, df = _sc_select(bboxes.reshape(_N * 4), labels)
    t0, t1, t2 = _tc_apply_call(di, df, z0, z1, z2)
    return (image, t0, t1, t2)


# speedup vs baseline: 1.1404x; 1.1404x over previous
import functools

import jax
import jax.numpy as jnp
import numpy as np
from jax import lax
from jax.experimental import pallas as pl
from jax.experimental.pallas import tpu as pltpu
from jax.experimental.pallas import tpu_sc as plsc

_S = (16, 32, 64)
_INV_SCALE = (1.0 / 32.0, 1.0 / 16.0, 1.0 / 8.0)
_C = 80
_N = 5000
_NSUB = 16
_VPS = 24
_NVT = _NSUB * _VPS
_ROW = 5 + _C
_BIG = 1 << 30

_ANCH = [
    [(np.float32(3.6), np.float32(2.8)), (np.float32(5.0), np.float32(6.2)),
     (np.float32(11.7), np.float32(10.2))],
    [(np.float32(1.9), np.float32(3.8)), (np.float32(3.9), np.float32(2.9)),
     (np.float32(3.7), np.float32(7.4))],
    [(np.float32(0.8), np.float32(1.0)), (np.float32(1.6), np.float32(2.2)),
     (np.float32(2.1), np.float32(4.4))],
]
_AREA = [[np.float32(a[0] * a[1]) for a in sc] for sc in _ANCH]

_GDN = lax.GatherDimensionNumbers(offset_dims=(), collapsed_slice_dims=(0,),
                                  start_index_map=(0,))


def _shuf(x, perm):
    return lax.gather(x, perm[:, None], _GDN, (1,),
                      mode=lax.GatherScatterMode.PROMISE_IN_BOUNDS)


def _vmin16(x):
    lanes = lax.iota(jnp.int32, 16)
    for sh in (8, 4, 2, 1):
        x = jnp.minimum(x, _shuf(x, (lanes + sh) & 15))
    return x


def _vor16(x):
    lanes = lax.iota(jnp.int32, 16)
    for sh in (8, 4, 2, 1):
        x = x | _shuf(x, (lanes + sh) & 15)
    return x


_LN2 = 0.6931471805599453
_SQRT2 = 1.4142135623730951


def _log1p6(x):
    bits = plsc.bitcast(x, jnp.int32)
    e = ((bits >> 23) & 255) - 127
    m = plsc.bitcast((bits & 0x007FFFFF) | 0x3F800000, jnp.float32)
    big = m > _SQRT2
    m = jnp.where(big, m * 0.5, m)
    ef = (e + jnp.where(big, 1, 0)).astype(jnp.float32)
    r = (m - 1.0) / (m + 1.0)
    r2 = r * r
    p = 1.0 + r2 * (1.0 / 3.0 + r2 * (0.2 + r2 * (1.0 / 7.0 + r2 * (1.0 / 9.0))))
    return 2.0 * r * p + ef * _LN2


_info = plsc.get_sparse_core_info()
_MESH = plsc.VectorSubcoreMesh(core_axis_name="c", subcore_axis_name="s")

_DI = 72
_DF = 36


@functools.partial(
    pl.kernel,
    mesh=_MESH,
    out_type=[jax.ShapeDtypeStruct((_DI,), jnp.int32),
              jax.ShapeDtypeStruct((_DF,), jnp.float32)],
    scratch_types=[
        pltpu.VMEM((_N * 4,), jnp.float32),
        pltpu.VMEM((_N,), jnp.int32),
        pltpu.VMEM((_NVT * 16,), jnp.int32),
        pltpu.VMEM((_NVT,), jnp.int32),
        pltpu.VMEM((_VPS * 16,), jnp.int32),
        pltpu.VMEM((_VPS,), jnp.int32),
        pltpu.VMEM_SHARED((_NVT * 16,), jnp.int32),
        pltpu.VMEM_SHARED((_NVT,), jnp.int32),
        pltpu.VMEM((_DI,), jnp.int32),
        pltpu.VMEM((_DF,), jnp.float32),
        pltpu.SemaphoreType.DMA,
    ],
    compiler_params=pltpu.CompilerParams(needs_layout_passes=False),
)
def _sc_select(bb_hbm, lab_hbm, di_out, df_out,
               vbb, vlab, vmask, vl1, vml, vl1l, shm, shl1, vdi, vdf, sem):
    nc = _info.num_cores
    sub = lax.axis_index("s")
    wid = sub * nc + lax.axis_index("c")
    lanes = lax.iota(jnp.int32, 16)
    zi = jnp.zeros((16,), jnp.int32)

    cps = [pltpu.async_copy(bb_hbm, vbb, sem),
           pltpu.async_copy(lab_hbm, vlab, sem)]
    for c in cps:
        c.wait()

    def mbody(t, _):
        idx = (sub * _VPS + t) * 16 + lanes
        rows = jnp.minimum(idx, _N - 1) * 4
        w = plsc.load_gather(vbb, [rows + 2])
        h = plsc.load_gather(vbb, [rows + 3])
        m = jnp.zeros((16,), jnp.int32)
        for s in range(3):
            bw = w * _INV_SCALE[s]
            bh = h * _INV_SCALE[s]
            area = bw * bh
            for a in range(3):
                aw, ah = _ANCH[s][a]
                inter = jnp.minimum(aw, bw) * jnp.minimum(ah, bh)
                iou = inter / (area + _AREA[s][a] - inter)
                m = m | jnp.where(iou >= 0.2, jnp.int32(1 << (3 * s + a)),
                                  jnp.int32(0))
        m = jnp.where(idx < _N, m, 0)
        vml[pl.ds(t * 16, 16)] = m
        plsc.store_scatter(vl1l, [jnp.minimum(zi + t, _VPS - 1)], _vor16(m),
                           mask=lanes == 0)
        return 0
    lax.fori_loop(0, _VPS, mbody, 0)

    pltpu.sync_copy(vml, shm.at[pl.ds(sub * (_VPS * 16), _VPS * 16)])
    pltpu.sync_copy(vl1l, shl1.at[pl.ds(sub * _VPS, _VPS)])
    plsc.subcore_barrier()
    pltpu.sync_copy(shm, vmask)
    pltpu.sync_copy(shl1, vl1)

    for s in range(3):
        grid = _S[s]
        inv = _INV_SCALE[s]
        sh3 = 3 * s
        allowed = jnp.full((16,), 7, jnp.int32)
        u_prev = jnp.full((16,), -1, jnp.int32)
        for k in range(3):
            def s1(t, acc):
                l1 = vl1[pl.ds(t * 16, 16)]
                jdx = t * 16 + lanes
                cand = (((l1 >> sh3) & allowed) != 0) & \
                       (jdx * 16 + 15 > u_prev)
                return jnp.minimum(acc, jnp.where(cand, jdx, _BIG))
            acc = lax.fori_loop(0, _NVT // 16, s1,
                                jnp.full((16,), _BIG, jnp.int32))
            j1 = _vmin16(acc)
            j1c = jnp.minimum(j1, _NVT - 1)
            gidx = j1c * 16 + lanes
            grp = plsc.load_gather(vmask, [gidx])
            cand = (((grp >> sh3) & allowed) != 0) & (gidx > u_prev)
            u_a = _vmin16(jnp.where(cand, gidx, _BIG))
            def s2(t, acc):
                l1 = vl1[pl.ds(t * 16, 16)]
                jdx = t * 16 + lanes
                cand = (((l1 >> sh3) & allowed) != 0) & (jdx > j1)
                return jnp.minimum(acc, jnp.where(cand, jdx, _BIG))
            acc2 = lax.fori_loop(0, _NVT // 16, s2,
                                 jnp.full((16,), _BIG, jnp.int32))
            j2c = jnp.minimum(_vmin16(acc2), _NVT - 1)
            gidx2 = j2c * 16 + lanes
            grp2 = plsc.load_gather(vmask, [gidx2])
            cand2 = (((grp2 >> sh3) & allowed) != 0) & (gidx2 > u_prev)
            u_b = _vmin16(jnp.where(cand2, gidx2, _BIG))
            u = jnp.where(u_a < _BIG, u_a, u_b)
            found = u < _BIG
            idxv = jnp.minimum(u, _N - 1)

            bw = plsc.load_gather(vbb, [idxv * 4 + 2]) * inv
            bh = plsc.load_gather(vbb, [idxv * 4 + 3]) * inv
            cxs = plsc.load_gather(vbb, [idxv * 4]) * inv
            cys = plsc.load_gather(vbb, [idxv * 4 + 1]) * inv
            labv = plsc.load_gather(vlab, [idxv])
            area = bw * bh
            miou = []
            for a in range(3):
                aw, ah = _ANCH[s][a]
                inter = jnp.minimum(aw, bw) * jnp.minimum(ah, bh)
                iou = inter / (area + _AREA[s][a] - inter)
                al = ((allowed >> a) & 1) != 0
                miou.append(jnp.where(al, iou, 0.0))
            best = jnp.where(miou[1] > miou[0], jnp.int32(1), jnp.int32(0))
            bm = jnp.maximum(miou[0], miou[1])
            best = jnp.where(miou[2] > bm, jnp.int32(2), best)
            best_iou = jnp.maximum(bm, miou[2])
            ig = [((miou[a] > 0.5) & (miou[a] != best_iou)).astype(jnp.int32)
                  for a in range(3)]

            px = cxs.astype(jnp.int32)
            py = cys.astype(jnp.int32)
            bx = cxs - px.astype(jnp.float32)
            by = cys - py.astype(jnp.float32)
            awv = jnp.where(best == 0, _ANCH[s][0][0],
                            jnp.where(best == 1, _ANCH[s][1][0], _ANCH[s][2][0]))
            ahv = jnp.where(best == 0, _ANCH[s][0][1],
                            jnp.where(best == 1, _ANCH[s][1][1], _ANCH[s][2][1]))
            lw = _log1p6(bw / awv + 1e-6)
            lh = _log1p6(bh / ahv + 1e-6)

            vi = jnp.where(
                lanes == 0, found.astype(jnp.int32),
                jnp.where(lanes == 1, px,
                          jnp.where(lanes == 2, py,
                                    jnp.where(lanes == 3, best,
                                              jnp.where(lanes == 4, labv,
                                                        jnp.where(lanes == 5, ig[0],
                                                                  jnp.where(lanes == 6,
                                                                            ig[1], ig[2])))))))
            bi = s * 24 + k * 8
            plsc.store_scatter(vdi, [jnp.minimum(bi + lanes, _DI - 1)], vi,
                               mask=lanes < 8)
            vf = jnp.where(lanes == 0, bx,
                           jnp.where(lanes == 1, by,
                                     jnp.where(lanes == 2, lw, lh)))
            bf = s * 12 + k * 4
            plsc.store_scatter(vdf, [jnp.minimum(bf + lanes, _DF - 1)], vf,
                               mask=lanes < 4)

            bbit = jnp.int32(1) << best
            allowed = jnp.where(found, allowed & ~bbit, allowed)
            u_prev = u

    @pl.when(wid == 0)
    def _():
        pltpu.sync_copy(vdi, di_out)
        pltpu.sync_copy(vdf, df_out)


def _tc_fill(di, df, o0, o1, o2):
    outs = (o0, o1, o2)
    for s in range(3):
        g = _S[s]
        outs[s][...] = jnp.zeros((g, g, 3, _ROW), jnp.float32)
    for s in range(3):
        o = outs[s]
        for k in range(3):
            b = s * 24 + k * 8
            fb = s * 12 + k * 4

            @pl.when(di[b] == 1)
            def _(o=o, b=b, fb=fb):
                px = di[b + 1]
                py = di[b + 2]
                bst = di[b + 3]
                lab = di[b + 4]
                c85 = lax.broadcasted_iota(jnp.int32, (1, 1, 1, _ROW), 3)
                row = jnp.where(
                    c85 == 0, 1.0,
                    jnp.where(c85 == 1, df[fb],
                              jnp.where(c85 == 2, df[fb + 1],
                                        jnp.where(c85 == 3, df[fb + 2],
                                                  jnp.where(c85 == 4, df[fb + 3],
                                                            jnp.where(c85 == 5 + lab,
                                                                      1.0, 0.0))))))
                o[pl.ds(px, 1), pl.ds(py, 1), pl.ds(bst, 1), :] = row
                cur = o[pl.ds(px, 1), pl.ds(py, 1), :, :]
                arow = lax.broadcasted_iota(jnp.int32, (1, 1, 3, _ROW), 2)
                ccol = lax.broadcasted_iota(jnp.int32, (1, 1, 3, _ROW), 3)
                igm = (((arow == 0) & (di[b + 5] == 1)) |
                       ((arow == 1) & (di[b + 6] == 1)) |
                       ((arow == 2) & (di[b + 7] == 1))) & (ccol == 0)
                o[pl.ds(px, 1), pl.ds(py, 1), :, :] = jnp.where(igm, -1.0, cur)


_tc_call = pl.pallas_call(
    _tc_fill,
    out_shape=[jax.ShapeDtypeStruct((g, g, 3, _ROW), jnp.float32) for g in _S],
    in_specs=[pl.BlockSpec(memory_space=pltpu.SMEM),
              pl.BlockSpec(memory_space=pltpu.SMEM)],
)


def kernel(image, bboxes, labels):
    di, df = _sc_select(bboxes.reshape(_N * 4), labels)
    t0, t1, t2 = _tc_call(di, df)
    return (image, t0, t1, t2)
